# baseline (speedup 1.0000x reference)
import jax
import jax.numpy as jnp
from jax import lax
from jax.experimental import pallas as pl
from jax.experimental.pallas import tpu as pltpu
from jax.experimental.pallas import tpu_sc as plsc

L, B, T, H, D = 8, 8, 64, 8, 64
S = 512
LB = L * B
HD = H * D
NC, NS = 2, 16
NW = NC * NS
ROWS_PER_W = LB // NW
ZPT = 32
NZTILES = (S - T) // ZPT


def _sc_body(k_hbm, v_hbm, kb_hbm, vb_hbm, zbuf, kstage, vstage,
             zspmem, sem, zsem):
    c = lax.axis_index("c")
    s = lax.axis_index("s")
    wid = s * NC + c
    zero16 = jnp.zeros((16,), jnp.float32)

    def zfill(t, carry):
        for q in range(HD // 16):
            zbuf[t, pl.ds(q * 16, 16)] = zero16
        return carry

    @pl.when(s < NZTILES)
    def _stage_zeros():
        lax.fori_loop(0, ZPT, zfill, 0)
        off = pl.multiple_of(s * ZPT, ZPT)
        pltpu.sync_copy(zbuf, zspmem.at[pl.ds(off, ZPT)])

    plsc.subcore_barrier()

    zcopies = []
    for rl in range(ROWS_PER_W):
        r = wid * ROWS_PER_W + rl
        zcopies.append(pltpu.async_copy(
            zspmem, kb_hbm.at[r, pl.ds(T, S - T)], zsem))
        zcopies.append(pltpu.async_copy(
            zspmem, vb_hbm.at[r, pl.ds(T, S - T)], zsem))

    for rl in range(ROWS_PER_W):
        r = wid * ROWS_PER_W + rl
        gk = pltpu.async_copy(k_hbm.at[r], kstage, sem)
        gv = pltpu.async_copy(v_hbm.at[r], vstage, sem)
        gk.wait()
        gv.wait()
        sk = pltpu.async_copy(kstage, kb_hbm.at[r, pl.ds(0, T)], sem)
        sv = pltpu.async_copy(vstage, vb_hbm.at[r, pl.ds(0, T)], sem)
        sk.wait()
        sv.wait()

    for zc in zcopies:
        zc.wait()


_sc_fill = pl.kernel(
    _sc_body,
    out_type=[
        jax.ShapeDtypeStruct((LB, S, HD), jnp.float32),
        jax.ShapeDtypeStruct((LB, S, HD), jnp.float32),
    ],
    mesh=plsc.VectorSubcoreMesh(core_axis_name="c", subcore_axis_name="s"),
    scratch_types=[
        pltpu.VMEM((ZPT, HD), jnp.float32),
        pltpu.VMEM((T, HD), jnp.float32),
        pltpu.VMEM((T, HD), jnp.float32),
        pltpu.VMEM_SHARED((S - T, HD), jnp.float32),
        pltpu.SemaphoreType.DMA,
        pltpu.SemaphoreType.DMA,
    ],
    compiler_params=pltpu.CompilerParams(use_tc_tiling_on_sc=True),
)


def _tc_body(k_ref, v_ref, ko_ref, vo_ref):
    ko_ref[...] = k_ref[...]
    vo_ref[...] = v_ref[...]


def _tc_out(k2, v2):
    n = LB * T
    blk = n // 8
    return pl.pallas_call(
        _tc_body,
        grid=(8,),
        in_specs=[
            pl.BlockSpec((blk, HD), lambda i: (i, 0)),
            pl.BlockSpec((blk, HD), lambda i: (i, 0)),
        ],
        out_specs=[
            pl.BlockSpec((blk, HD), lambda i: (i, 0)),
            pl.BlockSpec((blk, HD), lambda i: (i, 0)),
        ],
        out_shape=[
            jax.ShapeDtypeStruct((n, HD), jnp.float32),
            jax.ShapeDtypeStruct((n, HD), jnp.float32),
        ],
    )(k2, v2)


def kernel(keys, values, keys_buf, values_buf):
    kb, vb = _sc_fill(keys.reshape(LB, T, HD), values.reshape(LB, T, HD))
    ko, vo = _tc_out(keys.reshape(LB * T, HD), values.reshape(LB * T, HD))
    return (
        kb.reshape(keys_buf.shape),
        vb.reshape(values_buf.shape),
        ko.reshape(keys.shape),
        vo.reshape(values.shape),
    )


# speedup vs baseline: 1.7221x; 1.7221x over previous
import jax
import jax.numpy as jnp
from jax import lax
from jax.experimental import pallas as pl
from jax.experimental.pallas import tpu as pltpu
from jax.experimental.pallas import tpu_sc as plsc

L, B, T, H, D = 8, 8, 64, 8, 64
S = 512
LB = L * B
HD = H * D
NC, NS = 2, 16
NW = NC * NS
ROWS_PER_W = LB // NW
ZPT = S // NS


def _sc_zero_body(kb_hbm, vb_hbm, zbuf, zspmem, sem):
    c = lax.axis_index("c")
    s = lax.axis_index("s")
    wid = s * NC + c
    zero16 = jnp.zeros((16,), jnp.float32)

    def zfill(t, carry):
        for q in range(HD // 16):
            zbuf[t, pl.ds(q * 16, 16)] = zero16
        return carry

    lax.fori_loop(0, ZPT, zfill, 0)
    off = pl.multiple_of(s * ZPT, ZPT)
    pltpu.sync_copy(zbuf, zspmem.at[pl.ds(off, ZPT)])
    plsc.subcore_barrier()

    copies = []
    for rl in range(ROWS_PER_W):
        r = wid * ROWS_PER_W + rl
        copies.append(pltpu.async_copy(zspmem, kb_hbm.at[r], sem))
        copies.append(pltpu.async_copy(zspmem, vb_hbm.at[r], sem))
    for cp in copies:
        cp.wait()


_sc_zero = pl.kernel(
    _sc_zero_body,
    out_type=[
        jax.ShapeDtypeStruct((LB, HD, S), jnp.float32),
        jax.ShapeDtypeStruct((LB, HD, S), jnp.float32),
    ],
    mesh=plsc.VectorSubcoreMesh(core_axis_name="c", subcore_axis_name="s"),
    scratch_types=[
        pltpu.VMEM((ZPT, S), jnp.float32),
        pltpu.VMEM_SHARED((HD, S), jnp.float32),
        pltpu.SemaphoreType.DMA,
    ],
    compiler_params=pltpu.CompilerParams(use_tc_tiling_on_sc=True),
)


def _tc_out_body(k_ref, v_ref, ko_ref, vo_ref):
    ko_ref[...] = k_ref[...]
    vo_ref[...] = v_ref[...]


def _tc_out(keys, values):
    spec = pl.BlockSpec((1, 1, T, H, D), lambda i: (i // B, i % B, 0, 0, 0))
    return pl.pallas_call(
        _tc_out_body,
        grid=(LB,),
        in_specs=[spec, spec],
        out_specs=[spec, spec],
        out_shape=[
            jax.ShapeDtypeStruct((L, B, T, H, D), jnp.float32),
            jax.ShapeDtypeStruct((L, B, T, H, D), jnp.float32),
        ],
    )(keys, values)


def _tc_tok_body(k_ref, v_ref, kbz_ref, vbz_ref, kb_ref, vb_ref):
    k = k_ref[0, 0]
    v = v_ref[0, 0]
    zpad = jnp.zeros((HD, 2 * T - T), jnp.float32)
    kb_ref[0, :, T:] = zpad
    vb_ref[0, :, T:] = zpad
    for h in range(H):
        kb_ref[0, pl.ds(h * D, D), :T] = jnp.transpose(k[:, h, :])
        vb_ref[0, pl.ds(h * D, D), :T] = jnp.transpose(v[:, h, :])


def _tc_tok(keys, values, kbz, vbz):
    in5 = pl.BlockSpec((1, 1, T, H, D), lambda i: (i // B, i % B, 0, 0, 0))
    tok = pl.BlockSpec((1, HD, 2 * T), lambda i: (i, 0, 0))
    return pl.pallas_call(
        _tc_tok_body,
        grid=(LB,),
        in_specs=[in5, in5, tok, tok],
        out_specs=[tok, tok],
        out_shape=[
            jax.ShapeDtypeStruct((LB, HD, S), jnp.float32),
            jax.ShapeDtypeStruct((LB, HD, S), jnp.float32),
        ],
        input_output_aliases={2: 0, 3: 1},
    )(keys, values, kbz, vbz)


def kernel(keys, values, keys_buf, values_buf):
    kbz, vbz = _sc_zero()
    ko, vo = _tc_out(keys, values)
    kbp, vbp = _tc_tok(keys, values, kbz, vbz)
    kb = jnp.transpose(kbp.reshape(L, B, H, D, S), (0, 1, 4, 2, 3))
    vb = jnp.transpose(vbp.reshape(L, B, H, D, S), (0, 1, 4, 2, 3))
    return (kb, vb, ko, vo)
